# baseline (speedup 1.0000x reference)
import functools

import jax
import jax.numpy as jnp
import numpy as np
from jax import lax
from jax.experimental import pallas as pl
from jax.experimental.pallas import tpu as pltpu
from jax.experimental.pallas import tpu_sc as plsc

_B = 8
_NG = 8
_H = 128
_W = 128
_ETA_PX = 3.0
_POS_W = 1.2
_LAMBDA_CD = 1.0
_K_POS_CAP = 96
_LAMBDA_P0 = 1.0
_MAXI = 0x7FFFFFFF
_LO_BITS = int(np.float32(1e-13).view(np.int32))
_HI_BITS = int(np.float32(1048576.0).view(np.int32))

_NPLANES = _B * _NG
_NPIX = _H * _W
_LANES = 16
_NCHUNK = _NPIX // _LANES
_P1_SHIFT, _P1_NB = 19, 1024
_P2_SHIFT, _P2_NB = 9, 1024
_P3_NB = 512


def _log_sigmoid(x):
    return jnp.minimum(x, 0.0) - jnp.log1p(jnp.exp(-jnp.abs(x)))


def _seg_dist2(px, py, x1, y1, x2, y2):
    vx, vy = x2 - x1, y2 - y1
    wx, wy = px - x1, py - y1
    vv = vx * vx + vy * vy + 1e-09
    t = jnp.clip((wx * vx + wy * vy) * (1.0 / vv), 0.0, 1.0)
    dx = px - (x1 + t * vx)
    dy = py - (y1 + t * vy)
    return dx * dx + dy * dy + 1e-12


def _phase_a_kernel(gt_ref, s_ref, reg_ref, key_ref, loss_ref):
    b = pl.program_id(0)
    s = s_ref[0]

    iy = lax.broadcasted_iota(jnp.int32, (_H, _W), 0).astype(jnp.float32)
    ix = lax.broadcasted_iota(jnp.int32, (_H, _W), 1).astype(jnp.float32)
    yy = (iy + 0.5) * s
    xx = (ix + 0.5) * s

    for j in range(_NG):
        ax = gt_ref[b, j, 0, 0]
        ay = gt_ref[b, j, 0, 1]
        bx = gt_ref[b, j, 1, 0]
        by = gt_ref[b, j, 1, 1]
        cx = gt_ref[b, j, 2, 0]
        cy = gt_ref[b, j, 2, 1]

        def sign(x1, y1, x2, y2, x3, y3):
            return (x1 - x3) * (y2 - y3) - (x2 - x3) * (y1 - y3)

        d1 = sign(xx, yy, ax, ay, bx, by)
        d2s = sign(xx, yy, bx, by, cx, cy)
        d3 = sign(xx, yy, cx, cy, ax, ay)
        has_neg = (d1 < 0) | (d2s < 0) | (d3 < 0)
        has_pos = (d1 > 0) | (d2s > 0) | (d3 > 0)
        inside = ~(has_neg & has_pos)
        d2 = jnp.minimum(_seg_dist2(xx, yy, ax, ay, bx, by),
                         jnp.minimum(_seg_dist2(xx, yy, bx, by, cx, cy),
                                     _seg_dist2(xx, yy, cx, cy, ax, ay)))
        mask = inside | (d2 <= _ETA_PX * _ETA_PX)
        d2_bits = lax.bitcast_convert_type(d2, jnp.int32)
        key_ref[0, j] = jnp.where(mask, d2_bits, jnp.int32(_MAXI))

        g0x = (ax - xx) / s
        g0y = (ay - yy) / s
        p0 = (reg_ref[0, 0] - g0x) ** 2 + (reg_ref[0, 1] - g0y) ** 2

        g1x = (bx - xx) / s
        g1y = (by - yy) / s
        g2x = (cx - xx) / s
        g2y = (cy - yy) / s
        p1x, p1y = reg_ref[0, 2], reg_ref[0, 3]
        p2x, p2y = reg_ref[0, 4], reg_ref[0, 5]

        def pdist(px_, py_, gx_, gy_):
            return jnp.sqrt((px_ - gx_) ** 2 + (py_ - gy_) ** 2 + 1e-12)

        d11 = pdist(p1x, p1y, g1x, g1y)
        d12 = pdist(p1x, p1y, g2x, g2y)
        d21 = pdist(p2x, p2y, g1x, g1y)
        d22 = pdist(p2x, p2y, g2x, g2y)
        cd = (jnp.minimum(d11, d12) + jnp.minimum(d21, d22)
              + jnp.minimum(d11, d21) + jnp.minimum(d12, d22))
        loss_ref[0, j] = _LAMBDA_P0 * p0 + _LAMBDA_CD * cd


def _sc_threshold_body(keys_hbm, out_hbm, key_v, hist_v, res_v):
    wid = lax.axis_index("s") * 2 + lax.axis_index("c")
    lane = lax.broadcasted_iota(jnp.int32, (_LANES,), 0)

    def find_kth(k_target):
        def run_pass(nbuckets, bucket_of, valid_of, k_tgt):
            def zero_body(c, _):
                hist_v[pl.ds(c * _LANES, _LANES)] = jnp.zeros(
                    (_LANES,), jnp.int32)
                return 0
            lax.fori_loop(0, nbuckets, zero_body, 0)

            def scat_body(c, _):
                kv = key_v[pl.ds(c * _LANES, _LANES)]
                vv = jnp.maximum(kv - _LO_BITS, 0)
                bkt = bucket_of(vv)
                idx = bkt * _LANES + lane
                plsc.addupdate_scatter(
                    hist_v, [idx], jnp.ones((_LANES,), jnp.int32),
                    mask=valid_of(vv))
                return 0
            lax.fori_loop(0, _NCHUNK, scat_body, 0)

            def scan_body(c, carry):
                found, bkt, cbefore, run_tot = carry
                tot = jnp.zeros((_LANES,), jnp.int32)
                for l in range(_LANES):
                    tot = tot + plsc.load_gather(
                        hist_v, [(c * _LANES + lane) * _LANES + l])
                cum = plsc.cumsum(tot) + run_tot
                hit = cum >= k_tgt
                nhit = jnp.max(plsc.all_reduce_population_count(hit), axis=0)
                ffs = jnp.max(plsc.all_reduce_ffs(hit), axis=0)
                cb = jnp.max(jnp.where(hit, run_tot, cum), axis=0)
                upd = jnp.logical_and(found == 0, nhit > 0)
                bkt = jnp.where(upd, c * _LANES + ffs, bkt)
                cbefore = jnp.where(upd, cb, cbefore)
                found = jnp.where(nhit > 0, jnp.int32(1), found)
                run_tot = jnp.max(cum, axis=0)
                return (found, bkt, cbefore, run_tot)

            return lax.fori_loop(
                0, nbuckets // _LANES, scan_body,
                (jnp.int32(0), jnp.int32(0), jnp.int32(0), jnp.int32(0)))

        _, b1, c1, _ = run_pass(
            _P1_NB,
            lambda v: jnp.minimum(v >> _P1_SHIFT, _P1_NB - 1),
            lambda v: jnp.ones((_LANES,), jnp.bool_),
            k_target)
        k2 = k_target - c1
        _, b2, c2, _ = run_pass(
            _P2_NB,
            lambda v: (v >> _P2_SHIFT) & (_P2_NB - 1),
            lambda v: (v >> _P1_SHIFT) == b1,
            k2)
        k3 = k2 - c2
        pre2 = b1 * _P2_NB + b2
        _, b3, _, _ = run_pass(
            _P3_NB,
            lambda v: v & (_P3_NB - 1),
            lambda v: (v >> _P2_SHIFT) == pre2,
            k3)
        vbits = (b1 << _P1_SHIFT) | (b2 << _P2_SHIFT) | b3
        return vbits + _LO_BITS

    res = jnp.zeros((_LANES,), jnp.int32)
    for slot in range(2):
        p = wid * 2 + slot
        pltpu.sync_copy(keys_hbm.at[p], key_v)
        thr = find_kth(jnp.int32(_K_POS_CAP))
        res = jnp.where(lane == slot, thr, res)
    res_v[...] = res
    pltpu.sync_copy(res_v, out_hbm.at[wid])


def _sc_thresholds(keys):
    fn = functools.partial(
        pl.kernel,
        out_type=jax.ShapeDtypeStruct((_NPLANES // 2, _LANES), jnp.int32),
        mesh=plsc.VectorSubcoreMesh(core_axis_name="c",
                                    subcore_axis_name="s"),
        scratch_types=[
            pltpu.VMEM((_NPIX,), jnp.int32),
            pltpu.VMEM((_P1_NB * _LANES,), jnp.int32),
            pltpu.VMEM((_LANES,), jnp.int32),
        ],
        compiler_params=pltpu.CompilerParams(needs_layout_passes=False),
    )(_sc_threshold_body)
    return fn(keys)


def _phase_c_kernel(thr_ref, keys_ref, loss_ref, obj_ref, cls_ref, out_ref,
                    acc_ref):
    b = pl.program_id(0)

    @pl.when(b == 0)
    def _init():
        for i in range(5):
            acc_ref[i] = 0.0

    obj_t = jnp.zeros((_H, _W), jnp.float32)
    mf_sum = jnp.zeros((_H, _W), jnp.float32)
    reg_plane = jnp.zeros((_H, _W), jnp.float32)
    pos = jnp.float32(0.0)
    for j in range(_NG):
        kj = keys_ref[0, j]
        mask_j = kj != _MAXI
        npix_j = jnp.sum(mask_j.astype(jnp.int32))
        mf_j = (mask_j & ((npix_j <= _K_POS_CAP) | (kj <= thr_ref[b, j]))
                ).astype(jnp.float32)
        obj_t = jnp.maximum(obj_t, mf_j)
        mf_sum = mf_sum + mf_j
        reg_plane = reg_plane + mf_j * loss_ref[0, j]
        pos = pos + jnp.minimum(npix_j, _K_POS_CAP).astype(jnp.float32)

    x = obj_ref[0, 0]
    obj_l = jnp.sum(-(_POS_W * obj_t * _log_sigmoid(x)
                      + (1.0 - obj_t) * _log_sigmoid(-x)))
    cls_l = jnp.sum(mf_sum * (-_log_sigmoid(cls_ref[0, 0])))
    neg = jnp.float32(_H * _W) - jnp.sum((obj_t > 0.5).astype(jnp.float32))

    acc_ref[0] = acc_ref[0] + jnp.sum(reg_plane)
    acc_ref[1] = acc_ref[1] + obj_l
    acc_ref[2] = acc_ref[2] + cls_l
    acc_ref[3] = acc_ref[3] + pos
    acc_ref[4] = acc_ref[4] + neg

    @pl.when(b == _B - 1)
    def _finalize():
        pos_eps = jnp.maximum(acc_ref[3], 1.0)
        neg_eps = jnp.maximum(acc_ref[4], 1.0)
        out_ref[0] = (acc_ref[0] / pos_eps
                      + acc_ref[1] / (pos_eps + neg_eps)
                      + acc_ref[2] / pos_eps)


def kernel(pred_reg, pred_obj, pred_cls, gt_points, stride):
    s = jnp.asarray(stride, jnp.float32).reshape(1)
    keys, loss = pl.pallas_call(
        _phase_a_kernel,
        grid=(_B,),
        in_specs=[
            pl.BlockSpec(memory_space=pltpu.SMEM),
            pl.BlockSpec(memory_space=pltpu.SMEM),
            pl.BlockSpec((1, 6, _H, _W), lambda b: (b, 0, 0, 0)),
        ],
        out_specs=[
            pl.BlockSpec((1, _NG, _H, _W), lambda b: (b, 0, 0, 0)),
            pl.BlockSpec((1, _NG, _H, _W), lambda b: (b, 0, 0, 0)),
        ],
        out_shape=[
            jax.ShapeDtypeStruct((_B, _NG, _H, _W), jnp.int32),
            jax.ShapeDtypeStruct((_B, _NG, _H, _W), jnp.float32),
        ],
    )(gt_points, s, pred_reg)

    thr_rows = _sc_thresholds(keys.reshape(_NPLANES, _NPIX))
    thr = thr_rows[:, :2].reshape(_B, _NG)

    out = pl.pallas_call(
        _phase_c_kernel,
        grid=(_B,),
        in_specs=[
            pl.BlockSpec(memory_space=pltpu.SMEM),
            pl.BlockSpec((1, _NG, _H, _W), lambda b: (b, 0, 0, 0)),
            pl.BlockSpec((1, _NG, _H, _W), lambda b: (b, 0, 0, 0)),
            pl.BlockSpec((1, 1, _H, _W), lambda b: (b, 0, 0, 0)),
            pl.BlockSpec((1, 1, _H, _W), lambda b: (b, 0, 0, 0)),
        ],
        out_specs=pl.BlockSpec(memory_space=pltpu.SMEM),
        out_shape=jax.ShapeDtypeStruct((1,), jnp.float32),
        scratch_shapes=[pltpu.SMEM((5,), jnp.float32)],
    )(thr, keys, loss, pred_obj, pred_cls)
    return out[0]


# speedup vs baseline: 1.0872x; 1.0872x over previous
import functools

import jax
import jax.numpy as jnp
import numpy as np
from jax import lax
from jax.experimental import pallas as pl
from jax.experimental.pallas import tpu as pltpu
from jax.experimental.pallas import tpu_sc as plsc

_B = 8
_NG = 8
_H = 128
_W = 128
_ETA_PX = 3.0
_POS_W = 1.2
_LAMBDA_CD = 1.0
_K_POS_CAP = 96
_LAMBDA_P0 = 1.0
_MAXI = 0x7FFFFFFF
_LO_BITS = int(np.float32(1e-13).view(np.int32))
_HI_BITS = int(np.float32(1048576.0).view(np.int32))

_NPLANES = _B * _NG
_NPIX = _H * _W
_LANES = 16
_NCHUNK = _NPIX // _LANES
_P1_SHIFT, _P1_NB = 19, 1024
_P2_SHIFT, _P2_NB = 9, 1024
_P3_NB = 512


def _log_sigmoid(x):
    return jnp.minimum(x, 0.0) - jnp.log1p(jnp.exp(-jnp.abs(x)))


def _seg_dist2(px, py, x1, y1, x2, y2):
    vx, vy = x2 - x1, y2 - y1
    wx, wy = px - x1, py - y1
    vv = vx * vx + vy * vy + 1e-09
    t = jnp.clip((wx * vx + wy * vy) * (1.0 / vv), 0.0, 1.0)
    dx = px - (x1 + t * vx)
    dy = py - (y1 + t * vy)
    return dx * dx + dy * dy + 1e-12


def _phase_a_kernel(gt_ref, s_ref, reg_ref, key_ref, loss_ref):
    b = pl.program_id(0)
    s = s_ref[0]

    iy = lax.broadcasted_iota(jnp.int32, (_H, _W), 0).astype(jnp.float32)
    ix = lax.broadcasted_iota(jnp.int32, (_H, _W), 1).astype(jnp.float32)
    yy = (iy + 0.5) * s
    xx = (ix + 0.5) * s

    for j in range(_NG):
        ax = gt_ref[b, j, 0, 0]
        ay = gt_ref[b, j, 0, 1]
        bx = gt_ref[b, j, 1, 0]
        by = gt_ref[b, j, 1, 1]
        cx = gt_ref[b, j, 2, 0]
        cy = gt_ref[b, j, 2, 1]

        def sign(x1, y1, x2, y2, x3, y3):
            return (x1 - x3) * (y2 - y3) - (x2 - x3) * (y1 - y3)

        d1 = sign(xx, yy, ax, ay, bx, by)
        d2s = sign(xx, yy, bx, by, cx, cy)
        d3 = sign(xx, yy, cx, cy, ax, ay)
        has_neg = (d1 < 0) | (d2s < 0) | (d3 < 0)
        has_pos = (d1 > 0) | (d2s > 0) | (d3 > 0)
        inside = ~(has_neg & has_pos)
        d2 = jnp.minimum(_seg_dist2(xx, yy, ax, ay, bx, by),
                         jnp.minimum(_seg_dist2(xx, yy, bx, by, cx, cy),
                                     _seg_dist2(xx, yy, cx, cy, ax, ay)))
        mask = inside | (d2 <= _ETA_PX * _ETA_PX)
        d2_bits = lax.bitcast_convert_type(d2, jnp.int32)
        key_ref[0, j] = jnp.where(mask, d2_bits, jnp.int32(_MAXI))

        g0x = (ax - xx) / s
        g0y = (ay - yy) / s
        p0 = (reg_ref[0, 0] - g0x) ** 2 + (reg_ref[0, 1] - g0y) ** 2

        g1x = (bx - xx) / s
        g1y = (by - yy) / s
        g2x = (cx - xx) / s
        g2y = (cy - yy) / s
        p1x, p1y = reg_ref[0, 2], reg_ref[0, 3]
        p2x, p2y = reg_ref[0, 4], reg_ref[0, 5]

        def pdist(px_, py_, gx_, gy_):
            return jnp.sqrt((px_ - gx_) ** 2 + (py_ - gy_) ** 2 + 1e-12)

        d11 = pdist(p1x, p1y, g1x, g1y)
        d12 = pdist(p1x, p1y, g2x, g2y)
        d21 = pdist(p2x, p2y, g1x, g1y)
        d22 = pdist(p2x, p2y, g2x, g2y)
        cd = (jnp.minimum(d11, d12) + jnp.minimum(d21, d22)
              + jnp.minimum(d11, d21) + jnp.minimum(d12, d22))
        loss_ref[0, j] = _LAMBDA_P0 * p0 + _LAMBDA_CD * cd


def _sc_threshold_body(keys_hbm, out_hbm, key_v, hist_v, tot_v, res_v):
    wid = lax.axis_index("s") * 2 + lax.axis_index("c")
    lane = lax.broadcasted_iota(jnp.int32, (_LANES,), 0)

    def find_kth(k_target):
        def run_pass(nbuckets, bucket_of, valid_of, k_tgt):
            def zero_body(c, _):
                for u in range(8):
                    hist_v[pl.ds((c * 8 + u) * _LANES, _LANES)] = jnp.zeros(
                        (_LANES,), jnp.int32)
                return 0
            lax.fori_loop(0, nbuckets // 8, zero_body, 0)

            def scat_body(c, _):
                for u in range(8):
                    kv = key_v[pl.ds((c * 8 + u) * _LANES, _LANES)]
                    vv = jnp.maximum(kv - _LO_BITS, 0)
                    bkt = bucket_of(vv)
                    idx = bkt * _LANES + lane
                    plsc.addupdate_scatter(
                        hist_v, [idx], jnp.ones((_LANES,), jnp.int32),
                        mask=valid_of(vv))
                return 0
            lax.fori_loop(0, _NCHUNK // 8, scat_body, 0)

            nchunks = nbuckets // _LANES

            def tot_body(g, _):
                tvec = jnp.zeros((_LANES,), jnp.int32)
                for i in range(_LANES):
                    c = g * _LANES + i
                    tot = jnp.zeros((_LANES,), jnp.int32)
                    for l in range(_LANES):
                        tot = tot + plsc.load_gather(
                            hist_v, [(c * _LANES + lane) * _LANES + l])
                    csum = jnp.max(plsc.cumsum(tot), axis=0)
                    tvec = jnp.where(lane == i, csum, tvec)
                tot_v[pl.ds(g * _LANES, _LANES)] = tvec
                return 0
            lax.fori_loop(0, nchunks // _LANES, tot_body, 0)

            def chunk_body(g, carry):
                found, tgt_chunk, cbefore, run_tot = carry
                tvec = tot_v[pl.ds(g * _LANES, _LANES)]
                cum = plsc.cumsum(tvec) + run_tot
                hit = cum >= k_tgt
                nhit = jnp.max(plsc.all_reduce_population_count(hit), axis=0)
                ffs = jnp.max(plsc.all_reduce_ffs(hit), axis=0)
                cb = jnp.max(jnp.where(hit, run_tot, cum), axis=0)
                upd = jnp.logical_and(found == 0, nhit > 0)
                tgt_chunk = jnp.where(upd, g * _LANES + ffs, tgt_chunk)
                cbefore = jnp.where(upd, cb, cbefore)
                found = jnp.where(nhit > 0, jnp.int32(1), found)
                run_tot = jnp.max(cum, axis=0)
                return (found, tgt_chunk, cbefore, run_tot)

            found, tgt_chunk, cbefore, _ = lax.fori_loop(
                0, nchunks // _LANES, chunk_body,
                (jnp.int32(0), jnp.int32(0), jnp.int32(0), jnp.int32(0)))

            tot = jnp.zeros((_LANES,), jnp.int32)
            for l in range(_LANES):
                tot = tot + plsc.load_gather(
                    hist_v, [(tgt_chunk * _LANES + lane) * _LANES + l])
            cum = plsc.cumsum(tot) + cbefore
            hit = cum >= k_tgt
            ffs = jnp.max(plsc.all_reduce_ffs(hit), axis=0)
            cb = jnp.max(jnp.where(hit, cbefore, cum), axis=0)
            return (found, tgt_chunk * _LANES + ffs, cb, jnp.int32(0))

        _, b1, c1, _ = run_pass(
            _P1_NB,
            lambda v: jnp.minimum(v >> _P1_SHIFT, _P1_NB - 1),
            lambda v: jnp.ones((_LANES,), jnp.bool_),
            k_target)
        k2 = k_target - c1
        _, b2, c2, _ = run_pass(
            _P2_NB,
            lambda v: (v >> _P2_SHIFT) & (_P2_NB - 1),
            lambda v: (v >> _P1_SHIFT) == b1,
            k2)
        k3 = k2 - c2
        pre2 = b1 * _P2_NB + b2
        _, b3, _, _ = run_pass(
            _P3_NB,
            lambda v: v & (_P3_NB - 1),
            lambda v: (v >> _P2_SHIFT) == pre2,
            k3)
        vbits = (b1 << _P1_SHIFT) | (b2 << _P2_SHIFT) | b3
        return vbits + _LO_BITS

    def slot_body(slot, res):
        p = wid * 2 + slot
        pltpu.sync_copy(keys_hbm.at[p], key_v)
        thr = find_kth(jnp.int32(_K_POS_CAP))
        return jnp.where(lane == slot, thr, res)

    res = lax.fori_loop(0, 2, slot_body, jnp.zeros((_LANES,), jnp.int32))
    res_v[...] = res
    pltpu.sync_copy(res_v, out_hbm.at[wid])


def _sc_thresholds(keys):
    fn = functools.partial(
        pl.kernel,
        out_type=jax.ShapeDtypeStruct((_NPLANES // 2, _LANES), jnp.int32),
        mesh=plsc.VectorSubcoreMesh(core_axis_name="c",
                                    subcore_axis_name="s"),
        scratch_types=[
            pltpu.VMEM((_NPIX,), jnp.int32),
            pltpu.VMEM((_P1_NB * _LANES,), jnp.int32),
            pltpu.VMEM((_P1_NB // _LANES,), jnp.int32),
            pltpu.VMEM((_LANES,), jnp.int32),
        ],
        compiler_params=pltpu.CompilerParams(needs_layout_passes=False),
    )(_sc_threshold_body)
    return fn(keys)


def _phase_c_kernel(thr_ref, keys_ref, loss_ref, obj_ref, cls_ref, out_ref,
                    acc_ref):
    b = pl.program_id(0)

    @pl.when(b == 0)
    def _init():
        for i in range(5):
            acc_ref[i] = 0.0

    obj_t = jnp.zeros((_H, _W), jnp.float32)
    mf_sum = jnp.zeros((_H, _W), jnp.float32)
    reg_plane = jnp.zeros((_H, _W), jnp.float32)
    pos = jnp.float32(0.0)
    for j in range(_NG):
        kj = keys_ref[0, j]
        mask_j = kj != _MAXI
        npix_j = jnp.sum(mask_j.astype(jnp.int32))
        mf_j = (mask_j & ((npix_j <= _K_POS_CAP) | (kj <= thr_ref[b, j]))
                ).astype(jnp.float32)
        obj_t = jnp.maximum(obj_t, mf_j)
        mf_sum = mf_sum + mf_j
        reg_plane = reg_plane + mf_j * loss_ref[0, j]
        pos = pos + jnp.minimum(npix_j, _K_POS_CAP).astype(jnp.float32)

    x = obj_ref[0, 0]
    obj_l = jnp.sum(-(_POS_W * obj_t * _log_sigmoid(x)
                      + (1.0 - obj_t) * _log_sigmoid(-x)))
    cls_l = jnp.sum(mf_sum * (-_log_sigmoid(cls_ref[0, 0])))
    neg = jnp.float32(_H * _W) - jnp.sum((obj_t > 0.5).astype(jnp.float32))

    acc_ref[0] = acc_ref[0] + jnp.sum(reg_plane)
    acc_ref[1] = acc_ref[1] + obj_l
    acc_ref[2] = acc_ref[2] + cls_l
    acc_ref[3] = acc_ref[3] + pos
    acc_ref[4] = acc_ref[4] + neg

    @pl.when(b == _B - 1)
    def _finalize():
        pos_eps = jnp.maximum(acc_ref[3], 1.0)
        neg_eps = jnp.maximum(acc_ref[4], 1.0)
        out_ref[0] = (acc_ref[0] / pos_eps
                      + acc_ref[1] / (pos_eps + neg_eps)
                      + acc_ref[2] / pos_eps)


def kernel(pred_reg, pred_obj, pred_cls, gt_points, stride):
    s = jnp.asarray(stride, jnp.float32).reshape(1)
    keys, loss = pl.pallas_call(
        _phase_a_kernel,
        grid=(_B,),
        in_specs=[
            pl.BlockSpec(memory_space=pltpu.SMEM),
            pl.BlockSpec(memory_space=pltpu.SMEM),
            pl.BlockSpec((1, 6, _H, _W), lambda b: (b, 0, 0, 0)),
        ],
        out_specs=[
            pl.BlockSpec((1, _NG, _H, _W), lambda b: (b, 0, 0, 0)),
            pl.BlockSpec((1, _NG, _H, _W), lambda b: (b, 0, 0, 0)),
        ],
        out_shape=[
            jax.ShapeDtypeStruct((_B, _NG, _H, _W), jnp.int32),
            jax.ShapeDtypeStruct((_B, _NG, _H, _W), jnp.float32),
        ],
    )(gt_points, s, pred_reg)

    thr_rows = _sc_thresholds(keys.reshape(_NPLANES, _NPIX))
    thr = thr_rows[:, :2].reshape(_B, _NG)

    out = pl.pallas_call(
        _phase_c_kernel,
        grid=(_B,),
        in_specs=[
            pl.BlockSpec(memory_space=pltpu.SMEM),
            pl.BlockSpec((1, _NG, _H, _W), lambda b: (b, 0, 0, 0)),
            pl.BlockSpec((1, _NG, _H, _W), lambda b: (b, 0, 0, 0)),
            pl.BlockSpec((1, 1, _H, _W), lambda b: (b, 0, 0, 0)),
            pl.BlockSpec((1, 1, _H, _W), lambda b: (b, 0, 0, 0)),
        ],
        out_specs=pl.BlockSpec(memory_space=pltpu.SMEM),
        out_shape=jax.ShapeDtypeStruct((1,), jnp.float32),
        scratch_shapes=[pltpu.SMEM((5,), jnp.float32)],
    )(thr, keys, loss, pred_obj, pred_cls)
    return out[0]


# speedup vs baseline: 2.8545x; 2.6255x over previous
import jax
import jax.numpy as jnp
import numpy as np
from jax.experimental import pallas as pl
from jax.experimental.pallas import tpu as pltpu

_B = 8
_NG = 8
_H = 128
_W = 128
_ETA_PX = 3.0
_POS_W = 1.2
_LAMBDA_CD = 1.0
_K_POS_CAP = 96
_LAMBDA_P0 = 1.0
_IPS = 2
_NP = _IPS * _NG
_MAXI = 0x7FFFFFFF
_LO_BITS = int(np.float32(1e-13).view(np.int32))
_HI_BITS = int(np.float32(1048576.0).view(np.int32))
_BS_ITERS = int(np.ceil(np.log2(float(_HI_BITS - _LO_BITS))))


def _log_sigmoid(x):
    return jnp.minimum(x, 0.0) - jnp.log1p(jnp.exp(-jnp.abs(x)))


def _seg_dist2(px, py, x1, y1, x2, y2):
    vx, vy = x2 - x1, y2 - y1
    wx, wy = px - x1, py - y1
    vv = vx * vx + vy * vy + 1e-09
    t = jnp.clip((wx * vx + wy * vy) * (1.0 / vv), 0.0, 1.0)
    dx = px - (x1 + t * vx)
    dy = py - (y1 + t * vy)
    return dx * dx + dy * dy + 1e-12


def _loss_kernel(gt_ref, s_ref, reg_ref, obj_ref, cls_ref, out_ref,
                 acc_ref, key_ref, loss_ref):
    g = pl.program_id(0)
    s = s_ref[0]

    @pl.when(g == 0)
    def _init():
        for i in range(5):
            acc_ref[i] = 0.0

    iy = jax.lax.broadcasted_iota(jnp.int32, (_H, _W), 0).astype(jnp.float32)
    ix = jax.lax.broadcasted_iota(jnp.int32, (_H, _W), 1).astype(jnp.float32)
    yy = (iy + 0.5) * s
    xx = (ix + 0.5) * s

    for ii in range(_IPS):
        for j in range(_NG):
            p = ii * _NG + j
            b = g * _IPS + ii
            ax = gt_ref[b, j, 0, 0]
            ay = gt_ref[b, j, 0, 1]
            bx = gt_ref[b, j, 1, 0]
            by = gt_ref[b, j, 1, 1]
            cx = gt_ref[b, j, 2, 0]
            cy = gt_ref[b, j, 2, 1]

            def sign(x1, y1, x2, y2, x3, y3):
                return (x1 - x3) * (y2 - y3) - (x2 - x3) * (y1 - y3)

            d1 = sign(xx, yy, ax, ay, bx, by)
            d2 = sign(xx, yy, bx, by, cx, cy)
            d3 = sign(xx, yy, cx, cy, ax, ay)
            has_neg = (d1 < 0) | (d2 < 0) | (d3 < 0)
            has_pos = (d1 > 0) | (d2 > 0) | (d3 > 0)
            inside = ~(has_neg & has_pos)
            d2m = jnp.minimum(_seg_dist2(xx, yy, ax, ay, bx, by),
                              jnp.minimum(_seg_dist2(xx, yy, bx, by, cx, cy),
                                          _seg_dist2(xx, yy, cx, cy, ax, ay)))
            mask = inside | (d2m <= _ETA_PX * _ETA_PX)
            d2_bits = jax.lax.bitcast_convert_type(d2m, jnp.int32)
            key_ref[p] = jnp.where(mask, d2_bits, jnp.int32(_MAXI))

            g0x = (ax - xx) / s
            g0y = (ay - yy) / s
            p0 = ((reg_ref[ii, 0] - g0x) ** 2
                  + (reg_ref[ii, 1] - g0y) ** 2)

            g1x = (bx - xx) / s
            g1y = (by - yy) / s
            g2x = (cx - xx) / s
            g2y = (cy - yy) / s
            p1x, p1y = reg_ref[ii, 2], reg_ref[ii, 3]
            p2x, p2y = reg_ref[ii, 4], reg_ref[ii, 5]

            def pdist(px_, py_, gx_, gy_):
                return jnp.sqrt((px_ - gx_) ** 2 + (py_ - gy_) ** 2 + 1e-12)

            d11 = pdist(p1x, p1y, g1x, g1y)
            d12 = pdist(p1x, p1y, g2x, g2y)
            d21 = pdist(p2x, p2y, g1x, g1y)
            d22 = pdist(p2x, p2y, g2x, g2y)
            cd = (jnp.minimum(d11, d12) + jnp.minimum(d21, d22)
                  + jnp.minimum(d11, d21) + jnp.minimum(d12, d22))
            loss_ref[p] = _LAMBDA_P0 * p0 + _LAMBDA_CD * cd

    key3 = key_ref[...]
    mask3 = key3 != _MAXI

    def _cnt(x):
        part = jnp.sum(x.astype(jnp.int32), axis=1)
        return jnp.sum(part, axis=1, keepdims=True)

    npix2 = _cnt(mask3)

    def bs_body(_, carry):
        lo, hi = carry
        mid = lo + (hi - lo) // 2
        cnt = _cnt(key_ref[...] <= mid.reshape(_NP, 1, 1))
        take = cnt >= _K_POS_CAP
        return (jnp.where(take, lo, mid + 1), jnp.where(take, mid, hi))

    lo2, _hi = jax.lax.fori_loop(
        0, _BS_ITERS, bs_body,
        (jnp.full((_NP, 1), _LO_BITS, jnp.int32),
         jnp.full((_NP, 1), _HI_BITS, jnp.int32)))
    lo = lo2.reshape(_NP, 1, 1)
    npix3 = npix2.reshape(_NP, 1, 1)

    mf3 = ((key3 <= lo) | ((npix3 <= _K_POS_CAP) & mask3)).astype(jnp.float32)
    mf4 = mf3.reshape(_IPS, _NG, _H, _W)
    obj_t = jnp.max(mf4, axis=1)
    mf_sum = jnp.sum(mf4, axis=1)
    reg_l = jnp.sum(mf3 * loss_ref[...])
    pos = jnp.sum(jnp.minimum(npix3, _K_POS_CAP)).astype(jnp.float32)

    x = obj_ref[...].reshape(_IPS, _H, _W)
    obj_l = jnp.sum(-(_POS_W * obj_t * _log_sigmoid(x)
                      + (1.0 - obj_t) * _log_sigmoid(-x)))
    c = cls_ref[...].reshape(_IPS, _H, _W)
    cls_l = jnp.sum(mf_sum * (-_log_sigmoid(c)))
    neg = (jnp.float32(_IPS * _H * _W)
           - jnp.sum((obj_t > 0.5).astype(jnp.float32)))

    acc_ref[0] = acc_ref[0] + reg_l
    acc_ref[1] = acc_ref[1] + obj_l
    acc_ref[2] = acc_ref[2] + cls_l
    acc_ref[3] = acc_ref[3] + pos
    acc_ref[4] = acc_ref[4] + neg

    @pl.when(g == _B // _IPS - 1)
    def _finalize():
        pos_eps = jnp.maximum(acc_ref[3], 1.0)
        neg_eps = jnp.maximum(acc_ref[4], 1.0)
        out_ref[0] = (acc_ref[0] / pos_eps
                      + acc_ref[1] / (pos_eps + neg_eps)
                      + acc_ref[2] / pos_eps)


def kernel(pred_reg, pred_obj, pred_cls, gt_points, stride):
    s = jnp.asarray(stride, jnp.float32).reshape(1)
    out = pl.pallas_call(
        _loss_kernel,
        grid=(_B // _IPS,),
        in_specs=[
            pl.BlockSpec(memory_space=pltpu.SMEM),
            pl.BlockSpec(memory_space=pltpu.SMEM),
            pl.BlockSpec((_IPS, 6, _H, _W), lambda g: (g, 0, 0, 0)),
            pl.BlockSpec((_IPS, 1, _H, _W), lambda g: (g, 0, 0, 0)),
            pl.BlockSpec((_IPS, 1, _H, _W), lambda g: (g, 0, 0, 0)),
        ],
        out_specs=pl.BlockSpec(memory_space=pltpu.SMEM),
        out_shape=jax.ShapeDtypeStruct((1,), jnp.float32),
        scratch_shapes=[
            pltpu.SMEM((5,), jnp.float32),
            pltpu.VMEM((_NP, _H, _W), jnp.int32),
            pltpu.VMEM((_NP, _H, _W), jnp.float32),
        ],
    )(gt_points, s, pred_reg, pred_obj, pred_cls)
    return out[0]


# speedup vs baseline: 3.1088x; 1.0891x over previous
import jax
import jax.numpy as jnp
import numpy as np
from jax.experimental import pallas as pl
from jax.experimental.pallas import tpu as pltpu

_B = 8
_NG = 8
_H = 128
_W = 128
_ETA_PX = 3.0
_POS_W = 1.2
_LAMBDA_CD = 1.0
_K_POS_CAP = 96
_LAMBDA_P0 = 1.0
_IPS = 4
_NP = _IPS * _NG
_MAXI = 0x7FFFFFFF
_LO_BITS = int(np.float32(1e-13).view(np.int32))
_HI_BITS = int(np.float32(1048576.0).view(np.int32))
_BS_ITERS = int(np.ceil(np.log2(float(_HI_BITS - _LO_BITS))))


def _log_sigmoid(x):
    return jnp.minimum(x, 0.0) - jnp.log1p(jnp.exp(-jnp.abs(x)))


def _seg_dist2(px, py, x1, y1, x2, y2):
    vx, vy = x2 - x1, y2 - y1
    wx, wy = px - x1, py - y1
    vv = vx * vx + vy * vy + 1e-09
    t = jnp.clip((wx * vx + wy * vy) * (1.0 / vv), 0.0, 1.0)
    dx = px - (x1 + t * vx)
    dy = py - (y1 + t * vy)
    return dx * dx + dy * dy + 1e-12


def _loss_kernel(gt_ref, s_ref, reg_ref, obj_ref, cls_ref, out_ref,
                 acc_ref, key_ref, loss_ref):
    g = pl.program_id(0)
    s = s_ref[0]

    @pl.when(g == 0)
    def _init():
        for i in range(5):
            acc_ref[i] = 0.0

    iy = jax.lax.broadcasted_iota(jnp.int32, (_H, _W), 0).astype(jnp.float32)
    ix = jax.lax.broadcasted_iota(jnp.int32, (_H, _W), 1).astype(jnp.float32)
    yy = (iy + 0.5) * s
    xx = (ix + 0.5) * s

    for ii in range(_IPS):
        for j in range(_NG):
            p = ii * _NG + j
            b = g * _IPS + ii
            ax = gt_ref[b, j, 0, 0]
            ay = gt_ref[b, j, 0, 1]
            bx = gt_ref[b, j, 1, 0]
            by = gt_ref[b, j, 1, 1]
            cx = gt_ref[b, j, 2, 0]
            cy = gt_ref[b, j, 2, 1]

            def sign(x1, y1, x2, y2, x3, y3):
                return (x1 - x3) * (y2 - y3) - (x2 - x3) * (y1 - y3)

            d1 = sign(xx, yy, ax, ay, bx, by)
            d2 = sign(xx, yy, bx, by, cx, cy)
            d3 = sign(xx, yy, cx, cy, ax, ay)
            has_neg = (d1 < 0) | (d2 < 0) | (d3 < 0)
            has_pos = (d1 > 0) | (d2 > 0) | (d3 > 0)
            inside = ~(has_neg & has_pos)
            d2m = jnp.minimum(_seg_dist2(xx, yy, ax, ay, bx, by),
                              jnp.minimum(_seg_dist2(xx, yy, bx, by, cx, cy),
                                          _seg_dist2(xx, yy, cx, cy, ax, ay)))
            mask = inside | (d2m <= _ETA_PX * _ETA_PX)
            d2_bits = jax.lax.bitcast_convert_type(d2m, jnp.int32)
            key_ref[p] = jnp.where(mask, d2_bits, jnp.int32(_MAXI))

            g0x = (ax - xx) / s
            g0y = (ay - yy) / s
            p0 = ((reg_ref[ii, 0] - g0x) ** 2
                  + (reg_ref[ii, 1] - g0y) ** 2)

            g1x = (bx - xx) / s
            g1y = (by - yy) / s
            g2x = (cx - xx) / s
            g2y = (cy - yy) / s
            p1x, p1y = reg_ref[ii, 2], reg_ref[ii, 3]
            p2x, p2y = reg_ref[ii, 4], reg_ref[ii, 5]

            def pdist(px_, py_, gx_, gy_):
                return jnp.sqrt((px_ - gx_) ** 2 + (py_ - gy_) ** 2 + 1e-12)

            d11 = pdist(p1x, p1y, g1x, g1y)
            d12 = pdist(p1x, p1y, g2x, g2y)
            d21 = pdist(p2x, p2y, g1x, g1y)
            d22 = pdist(p2x, p2y, g2x, g2y)
            cd = (jnp.minimum(d11, d12) + jnp.minimum(d21, d22)
                  + jnp.minimum(d11, d21) + jnp.minimum(d12, d22))
            loss_ref[p] = _LAMBDA_P0 * p0 + _LAMBDA_CD * cd

    key3 = key_ref[...]
    mask3 = key3 != _MAXI

    def _cnt(x):
        part = jnp.sum(x.astype(jnp.int32), axis=1)
        return jnp.sum(part, axis=1, keepdims=True)

    npix2 = _cnt(mask3)

    def bs_body(_, carry):
        lo, hi = carry
        mid = lo + (hi - lo) // 2
        cnt = _cnt(key_ref[...] <= mid.reshape(_NP, 1, 1))
        take = cnt >= _K_POS_CAP
        return (jnp.where(take, lo, mid + 1), jnp.where(take, mid, hi))

    lo2, _hi = jax.lax.fori_loop(
        0, _BS_ITERS, bs_body,
        (jnp.full((_NP, 1), _LO_BITS, jnp.int32),
         jnp.full((_NP, 1), _HI_BITS, jnp.int32)))
    lo = lo2.reshape(_NP, 1, 1)
    npix3 = npix2.reshape(_NP, 1, 1)

    mf3 = ((key3 <= lo) | ((npix3 <= _K_POS_CAP) & mask3)).astype(jnp.float32)
    mf4 = mf3.reshape(_IPS, _NG, _H, _W)
    obj_t = jnp.max(mf4, axis=1)
    mf_sum = jnp.sum(mf4, axis=1)
    reg_l = jnp.sum(mf3 * loss_ref[...])
    pos = jnp.sum(jnp.minimum(npix3, _K_POS_CAP)).astype(jnp.float32)

    x = obj_ref[...].reshape(_IPS, _H, _W)
    obj_l = jnp.sum(-(_POS_W * obj_t * _log_sigmoid(x)
                      + (1.0 - obj_t) * _log_sigmoid(-x)))
    c = cls_ref[...].reshape(_IPS, _H, _W)
    cls_l = jnp.sum(mf_sum * (-_log_sigmoid(c)))
    neg = (jnp.float32(_IPS * _H * _W)
           - jnp.sum((obj_t > 0.5).astype(jnp.float32)))

    acc_ref[0] = acc_ref[0] + reg_l
    acc_ref[1] = acc_ref[1] + obj_l
    acc_ref[2] = acc_ref[2] + cls_l
    acc_ref[3] = acc_ref[3] + pos
    acc_ref[4] = acc_ref[4] + neg

    @pl.when(g == _B // _IPS - 1)
    def _finalize():
        pos_eps = jnp.maximum(acc_ref[3], 1.0)
        neg_eps = jnp.maximum(acc_ref[4], 1.0)
        out_ref[0] = (acc_ref[0] / pos_eps
                      + acc_ref[1] / (pos_eps + neg_eps)
                      + acc_ref[2] / pos_eps)


def kernel(pred_reg, pred_obj, pred_cls, gt_points, stride):
    s = jnp.asarray(stride, jnp.float32).reshape(1)
    out = pl.pallas_call(
        _loss_kernel,
        grid=(_B // _IPS,),
        in_specs=[
            pl.BlockSpec(memory_space=pltpu.SMEM),
            pl.BlockSpec(memory_space=pltpu.SMEM),
            pl.BlockSpec((_IPS, 6, _H, _W), lambda g: (g, 0, 0, 0)),
            pl.BlockSpec((_IPS, 1, _H, _W), lambda g: (g, 0, 0, 0)),
            pl.BlockSpec((_IPS, 1, _H, _W), lambda g: (g, 0, 0, 0)),
        ],
        out_specs=pl.BlockSpec(memory_space=pltpu.SMEM),
        out_shape=jax.ShapeDtypeStruct((1,), jnp.float32),
        scratch_shapes=[
            pltpu.SMEM((5,), jnp.float32),
            pltpu.VMEM((_NP, _H, _W), jnp.int32),
            pltpu.VMEM((_NP, _H, _W), jnp.float32),
        ],
    )(gt_points, s, pred_reg, pred_obj, pred_cls)
    return out[0]
